# baseline (speedup 1.0000x reference)
import jax
import jax.numpy as jnp
from jax import lax
from jax.experimental import pallas as pl
from jax.experimental.pallas import tpu as pltpu
from jax.experimental.pallas import tpu_sc as plsc

K = 20000
NW = 32
ROWS_PER_W = 640
TAIL_ROWS = K - (NW - 1) * ROWS_PER_W
CHUNK = 128


def _body(ti_hbm, tabg3_hbm, out_hbm, ti_v, idx2_v, idxg_v, rows_v, out_v, sem):
    nc = 2
    wid = lax.axis_index("s") * nc + lax.axis_index("c")
    base = wid * ROWS_PER_W
    tabg_hbm = tabg3_hbm.at[0]
    pltpu.sync_copy(ti_hbm.at[0], ti_v)
    lanes = lax.iota(jnp.int32, 16)
    quarter = lanes >> 2
    comp = lanes & 3

    def compose(j, carry):
        first = ti_v[pl.ds(base + j * 16, 16)]
        sec = plsc.load_gather(ti_v, [first])
        idx2_v[pl.ds(j * 16, 16)] = sec
        idxg_v[j >> 3, pl.ds((j & 7) * 16, 16)] = sec >> 2
        return carry

    def extract(j, carry):
        row_local = j * 4 + quarter
        full = plsc.load_gather(idx2_v, [row_local])
        out_v[j] = plsc.load_gather(
            rows_v, [row_local, (full & 3) * 4 + comp]
        )
        return carry

    @pl.when(wid < NW - 1)
    def _full():
        lax.fori_loop(0, ROWS_PER_W // 16, compose, 0, unroll=4)
        cps = [
            pltpu.async_copy(
                tabg_hbm.at[idxg_v.at[t]],
                rows_v.at[pl.ds(t * CHUNK, CHUNK)],
                sem,
            )
            for t in range(ROWS_PER_W // CHUNK)
        ]
        for cp in cps:
            cp.wait()
        lax.fori_loop(0, ROWS_PER_W // 4, extract, 0, unroll=4)
        pltpu.sync_copy(
            out_v, out_hbm.at[0, pl.ds(wid * (ROWS_PER_W // 4), ROWS_PER_W // 4)]
        )

    @pl.when(wid == NW - 1)
    def _tail():
        lax.fori_loop(0, TAIL_ROWS // 16, compose, 0, unroll=4)
        cps = [
            pltpu.async_copy(
                tabg_hbm.at[idxg_v.at[0]], rows_v.at[pl.ds(0, CHUNK)], sem
            ),
            pltpu.async_copy(
                tabg_hbm.at[idxg_v.at[1, pl.ds(0, TAIL_ROWS - CHUNK)]],
                rows_v.at[pl.ds(CHUNK, TAIL_ROWS - CHUNK)],
                sem,
            ),
        ]
        for cp in cps:
            cp.wait()
        lax.fori_loop(0, TAIL_ROWS // 4, extract, 0, unroll=4)
        pltpu.sync_copy(
            out_v.at[pl.ds(0, TAIL_ROWS // 4)],
            out_hbm.at[0, pl.ds(wid * (ROWS_PER_W // 4), TAIL_ROWS // 4)],
        )


@jax.jit
def _run(ti, tab):
    mesh = plsc.VectorSubcoreMesh(
        core_axis_name="c", subcore_axis_name="s", num_cores=2, num_subcores=16
    )
    f = pl.kernel(
        _body,
        out_type=jax.ShapeDtypeStruct((1, K // 4, 16), jnp.float32),
        mesh=mesh,
        scratch_types=[
            pltpu.VMEM((K,), jnp.int32),
            pltpu.VMEM((ROWS_PER_W,), jnp.int32),
            pltpu.VMEM((ROWS_PER_W // CHUNK, CHUNK), jnp.int32),
            pltpu.VMEM((ROWS_PER_W, 16), jnp.float32),
            pltpu.VMEM((ROWS_PER_W // 4, 16), jnp.float32),
            pltpu.SemaphoreType.DMA,
        ],
        compiler_params=pltpu.CompilerParams(
            needs_layout_passes=False, use_tc_tiling_on_sc=False
        ),
    )
    return f(ti, tab)


def kernel(batch_idx, topk_idx, box_regression):
    tabg3 = box_regression.reshape(1, 40800 // 4, 16)
    return _run(topk_idx.astype(jnp.int32), tabg3).reshape(1, K, 4)


# speedup vs baseline: 2.2437x; 2.2437x over previous
import jax
import jax.numpy as jnp
from jax import lax
from jax.experimental import pallas as pl
from jax.experimental.pallas import tpu as pltpu
from jax.experimental.pallas import tpu_sc as plsc

K = 20000
NW = 32
ROWS_PER_W = 640
TAIL_ROWS = K - (NW - 1) * ROWS_PER_W


def _body(ti_hbm, tabt_hbm, out_hbm, ti_v, out_v, sem):
    nc = 2
    wid = lax.axis_index("s") * nc + lax.axis_index("c")
    base = wid * ROWS_PER_W

    def inner(tab_v):
        tab_cp = pltpu.async_copy(tabt_hbm, tab_v, sem)
        pltpu.sync_copy(ti_hbm.at[0], ti_v)

        def step(j, carry):
            first = ti_v[pl.ds(base + j * 16, 16)]
            idx2 = plsc.load_gather(ti_v, [first])
            for c in range(4):
                cc = jnp.full((16,), c, jnp.int32)
                out_v[c, pl.ds(j * 16, 16)] = plsc.load_gather(
                    tab_v, [cc, idx2]
                )
            return carry

        @pl.when(wid < NW - 1)
        def _full():
            tab_cp.wait()
            lax.fori_loop(0, ROWS_PER_W // 16, step, 0, unroll=4)
            for c in range(4):
                pltpu.sync_copy(
                    out_v.at[c], out_hbm.at[c, pl.ds(base, ROWS_PER_W)]
                )

        @pl.when(wid == NW - 1)
        def _tail():
            tab_cp.wait()
            lax.fori_loop(0, TAIL_ROWS // 16, step, 0, unroll=4)
            for c in range(4):
                pltpu.sync_copy(
                    out_v.at[c, pl.ds(0, TAIL_ROWS)],
                    out_hbm.at[c, pl.ds(base, TAIL_ROWS)],
                )

    pl.run_scoped(inner, pltpu.VMEM((4, K), jnp.float32))


@jax.jit
def _run(ti, tabt):
    mesh = plsc.VectorSubcoreMesh(
        core_axis_name="c", subcore_axis_name="s", num_cores=2, num_subcores=16
    )
    f = pl.kernel(
        _body,
        out_type=jax.ShapeDtypeStruct((4, K), jnp.float32),
        mesh=mesh,
        scratch_types=[
            pltpu.VMEM((K,), jnp.int32),
            pltpu.VMEM((4, ROWS_PER_W), jnp.float32),
            pltpu.SemaphoreType.DMA,
        ],
        compiler_params=pltpu.CompilerParams(
            needs_layout_passes=False, use_tc_tiling_on_sc=False
        ),
    )
    return f(ti, tabt)


def kernel(batch_idx, topk_idx, box_regression):
    tabt = box_regression[0, :K, :].T
    out = _run(topk_idx.astype(jnp.int32), tabt)
    return out.T[None]
